# Initial kernel scaffold; baseline (speedup 1.0000x reference)
import jax
import jax.numpy as jnp
from jax.experimental import pallas as pl


def kernel(center, context, rand, emb_table, lin_w):
    raise NotImplementedError("write your pallas kernel here")


# speedup vs baseline: 5.7969x; 5.7969x over previous
import functools

import jax
import jax.numpy as jnp
from jax import lax
from jax.experimental import pallas as pl
from jax.experimental.pallas import tpu as pltpu
from jax.experimental.pallas import tpu_sc as plsc

VOC = 100000
EMB = 128
B = 4096
C = 5
R = 10
NCR = C + R

NW = 32
BPW = B // NW
SUB = 16
NSUB = BPW // SUB
IDXS = SUB * NCR
HALF = IDXS // 2


def _sc_dots(center, idx_all, emb_table, lin_w):
    mesh = plsc.VectorSubcoreMesh(core_axis_name="c", subcore_axis_name="s")

    @functools.partial(
        pl.kernel,
        mesh=mesh,
        out_type=jax.ShapeDtypeStruct((B, 16), jnp.float32),
        scratch_types=[
            pltpu.VMEM((BPW,), jnp.int32),
            pltpu.VMEM((BPW * NCR,), jnp.int32),
            pltpu.VMEM((BPW, EMB), jnp.float32),
            pltpu.VMEM((IDXS, EMB), jnp.float32),
            pltpu.VMEM((BPW, 16), jnp.float32),
            pltpu.SemaphoreType.DMA,
        ],
    )
    def k(center_h, idx_h, emb_h, lin_h, out_h,
          cidx_v, widx_v, erows_v, wrows_v, dots_v, sem):
        cid = lax.axis_index("c")
        sid = lax.axis_index("s")
        wid = sid * 2 + cid
        base = wid * BPW

        pltpu.sync_copy(center_h.at[pl.ds(base, BPW)], cidx_v)
        pltpu.sync_copy(idx_h.at[pl.ds(wid * (BPW * NCR), BPW * NCR)], widx_v)
        pltpu.async_copy(emb_h.at[cidx_v], erows_v, sem).wait()

        lanes = lax.iota(jnp.int32, 16)
        perms = [lanes ^ jnp.int32(1 << p) for p in (3, 2, 1, 0)]

        def sub_body(s, carry):
            off = pl.multiple_of(s * IDXS, 8)
            cp1 = pltpu.async_copy(
                lin_h.at[widx_v.at[pl.ds(off, HALF)]],
                wrows_v.at[pl.ds(0, HALF)], sem)
            cp2 = pltpu.async_copy(
                lin_h.at[widx_v.at[pl.ds(off + HALF, HALF)]],
                wrows_v.at[pl.ds(HALF, HALF)], sem)
            cp1.wait()
            cp2.wait()

            def b_body(bb, carry2):
                b = s * SUB + bb
                e = [erows_v[b, pl.ds(16 * t, 16)] for t in range(8)]
                res = jnp.zeros((16,), jnp.float32)
                for j in range(NCR):
                    r = bb * NCR + j
                    acc = e[0] * wrows_v[r, pl.ds(0, 16)]
                    for t in range(1, 8):
                        acc = acc + e[t] * wrows_v[r, pl.ds(16 * t, 16)]
                    for p in perms:
                        acc = acc + jnp.take_along_axis(acc, p, axis=0)
                    res = jnp.where(lanes == j, acc, res)
                dots_v[b, :] = res
                return carry2

            lax.fori_loop(0, SUB, b_body, 0)
            return carry

        lax.fori_loop(0, NSUB, sub_body, 0)
        pltpu.sync_copy(dots_v, out_h.at[pl.ds(base, BPW)])

    return k(center, idx_all, emb_table, lin_w)


def _tc_loss(dots):
    def body(d_ref, o_ref):
        d = d_ref[...]
        col = lax.broadcasted_iota(jnp.int32, (B, 16), 1)
        sig = jax.nn.sigmoid(d)
        pos = -jnp.log(sig) * (1.0 / (B * C))
        neg = -jnp.log(1.0 - sig + 1e-3) * (1.0 / (B * R))
        val = jnp.where(col < C, pos, jnp.where(col < NCR, neg, 0.0))
        o_ref[0, 0] = jnp.sum(val)

    return pl.pallas_call(
        body,
        out_shape=jax.ShapeDtypeStruct((1, 1), jnp.float32),
        in_specs=[pl.BlockSpec((B, 16), lambda: (0, 0))],
        out_specs=pl.BlockSpec(memory_space=pltpu.SMEM),
    )(dots)


def kernel(center, context, rand, emb_table, lin_w):
    center = center.astype(jnp.int32)
    idx_all = jnp.concatenate(
        [context.astype(jnp.int32), rand.astype(jnp.int32)], axis=1
    ).reshape(B * NCR)
    dots = _sc_dots(center, idx_all, emb_table, lin_w)
    return _tc_loss(dots)[0, 0]


# speedup vs baseline: 6.8010x; 1.1732x over previous
import functools

import jax
import jax.numpy as jnp
from jax import lax
from jax.experimental import pallas as pl
from jax.experimental.pallas import tpu as pltpu
from jax.experimental.pallas import tpu_sc as plsc

VOC = 100000
EMB = 128
B = 4096
C = 5
R = 10
NCR = C + R

NW = 32
BPW = B // NW
SUB = 16
NSUB = BPW // SUB
IDXS = SUB * NCR
HALF = IDXS // 2


def _sc_dots(center, idx_all, emb_table, lin_w):
    mesh = plsc.VectorSubcoreMesh(core_axis_name="c", subcore_axis_name="s")

    @functools.partial(
        pl.kernel,
        mesh=mesh,
        out_type=jax.ShapeDtypeStruct((B, 16), jnp.float32),
        scratch_types=[
            pltpu.VMEM((BPW,), jnp.int32),
            pltpu.VMEM((BPW * NCR,), jnp.int32),
            pltpu.VMEM((BPW, EMB), jnp.float32),
            pltpu.VMEM((IDXS, EMB), jnp.float32),
            pltpu.VMEM((IDXS, EMB), jnp.float32),
            pltpu.VMEM((BPW, 16), jnp.float32),
            pltpu.SemaphoreType.DMA,
            pltpu.SemaphoreType.DMA,
            pltpu.SemaphoreType.DMA,
        ],
    )
    def k(center_h, idx_h, emb_h, lin_h, out_h,
          cidx_v, widx_v, erows_v, wrows0_v, wrows1_v, dots_v,
          esem, sem0, sem1):
        cid = lax.axis_index("c")
        sid = lax.axis_index("s")
        wid = sid * 2 + cid
        base = wid * BPW

        pltpu.sync_copy(center_h.at[pl.ds(base, BPW)], cidx_v)
        ecp = pltpu.async_copy(emb_h.at[cidx_v], erows_v, esem)
        pltpu.sync_copy(idx_h.at[pl.ds(wid * (BPW * NCR), BPW * NCR)], widx_v)

        wrows = (wrows0_v, wrows1_v)
        sems = (sem0, sem1)

        def issue(s):
            buf, sem = wrows[s % 2], sems[s % 2]
            off = s * IDXS
            return (
                pltpu.async_copy(lin_h.at[widx_v.at[pl.ds(off, HALF)]],
                                 buf.at[pl.ds(0, HALF)], sem),
                pltpu.async_copy(lin_h.at[widx_v.at[pl.ds(off + HALF, HALF)]],
                                 buf.at[pl.ds(HALF, HALF)], sem),
            )

        lanes = lax.iota(jnp.int32, 16)
        perms = [lanes ^ jnp.int32(1 << p) for p in (3, 2, 1, 0)]

        cps = issue(0)
        ecp.wait()
        for s in range(NSUB):
            nxt = issue(s + 1) if s + 1 < NSUB else None
            cps[0].wait()
            cps[1].wait()
            wbuf = wrows[s % 2]

            def b_body(bb, carry2, s=s, wbuf=wbuf):
                b = s * SUB + bb
                e = [erows_v[b, pl.ds(16 * t, 16)] for t in range(8)]
                res = jnp.zeros((16,), jnp.float32)
                for j in range(NCR):
                    r = bb * NCR + j
                    acc = e[0] * wbuf[r, pl.ds(0, 16)]
                    for t in range(1, 8):
                        acc = acc + e[t] * wbuf[r, pl.ds(16 * t, 16)]
                    for p in perms:
                        acc = acc + jnp.take_along_axis(acc, p, axis=0)
                    res = jnp.where(lanes == j, acc, res)
                dots_v[b, :] = res
                return carry2

            lax.fori_loop(0, SUB, b_body, 0)
            cps = nxt
        pltpu.sync_copy(dots_v, out_h.at[pl.ds(base, BPW)])

    return k(center, idx_all, emb_table, lin_w)


def _tc_loss(dots):
    def body(d_ref, o_ref):
        d = d_ref[...]
        col = lax.broadcasted_iota(jnp.int32, (B, 16), 1)
        sig = jax.nn.sigmoid(d)
        pos = -jnp.log(sig) * (1.0 / (B * C))
        neg = -jnp.log(1.0 - sig + 1e-3) * (1.0 / (B * R))
        val = jnp.where(col < C, pos, jnp.where(col < NCR, neg, 0.0))
        o_ref[0, 0] = jnp.sum(val)

    return pl.pallas_call(
        body,
        out_shape=jax.ShapeDtypeStruct((1, 1), jnp.float32),
        in_specs=[pl.BlockSpec((B, 16), lambda: (0, 0))],
        out_specs=pl.BlockSpec(memory_space=pltpu.SMEM),
    )(dots)


def kernel(center, context, rand, emb_table, lin_w):
    center = center.astype(jnp.int32)
    idx_all = jnp.concatenate(
        [context.astype(jnp.int32), rand.astype(jnp.int32)], axis=1
    ).reshape(B * NCR)
    dots = _sc_dots(center, idx_all, emb_table, lin_w)
    return _tc_loss(dots)[0, 0]
